# Initial kernel scaffold; baseline (speedup 1.0000x reference)
import jax
import jax.numpy as jnp
from jax.experimental import pallas as pl


def kernel(x, adj, W, b):
    raise NotImplementedError("write your pallas kernel here")


# speedup vs baseline: 6.9392x; 6.9392x over previous
import functools

import jax
import jax.numpy as jnp
from jax import lax
from jax.experimental import pallas as pl
from jax.experimental.pallas import tpu as pltpu
from jax.experimental.pallas import tpu_sc as plsc

N = 10000
IN_F = 128
F = 128
K = 32
L = 16
NC, NS = 2, 16
NW = NC * NS
RPW = -(-N // NW)
NCHUNK = N // L
H1_BITS, H2_BITS, H3_BITS = 11, 11, 8
H1_SIZE, H2_SIZE, H3_SIZE = 1 << H1_BITS, 1 << H2_BITS, 1 << H3_BITS
MM_BLOCK = 400


def _mm_body(x_ref, w_ref, b_ref, o_ref):
    o_ref[...] = lax.dot_general(
        x_ref[...], w_ref[...], (((1,), (1,)), ((), ())),
        preferred_element_type=jnp.float32) + b_ref[...]


_matmul = pl.pallas_call(
    _mm_body,
    out_shape=jax.ShapeDtypeStruct((N, F), jnp.float32),
    grid=(N // MM_BLOCK,),
    in_specs=[
        pl.BlockSpec((MM_BLOCK, IN_F), lambda i: (i, 0)),
        pl.BlockSpec((F, IN_F), lambda i: (0, 0)),
        pl.BlockSpec((1, F), lambda i: (0, 0)),
    ],
    out_specs=pl.BlockSpec((MM_BLOCK, F), lambda i: (i, 0)),
)


def _smax(v):
    return jnp.max(v)


_mesh = plsc.VectorSubcoreMesh(core_axis_name="c", subcore_axis_name="s")


@functools.partial(
    pl.kernel,
    out_type=jax.ShapeDtypeStruct((N, F), jnp.float32),
    mesh=_mesh,
    compiler_params=pltpu.CompilerParams(needs_layout_passes=False),
    scratch_types=[
        pltpu.VMEM((N,), jnp.float32),
        pltpu.VMEM((H1_SIZE + L,), jnp.int32),
        pltpu.VMEM((N + L,), jnp.int32),
        pltpu.VMEM((N + L,), jnp.int32),
        pltpu.VMEM((N + L,), jnp.int32),
        pltpu.VMEM((K + L,), jnp.int32),
        pltpu.VMEM((K,), jnp.int32),
        pltpu.VMEM((K, F), jnp.float32),
        pltpu.VMEM((F,), jnp.float32),
        pltpu.SemaphoreType.DMA,
    ],
)
def _sc_topk(adj_hbm, h_hbm, out_hbm, rowbuf, hist, cbits, cidx, eqidx,
             fpad, fidx, grows, orow, sem):
    wid = lax.axis_index("s") * NC + lax.axis_index("c")
    base = wid * RPW
    nrows = jnp.minimum(RPW, N - base)
    ones = jnp.ones((L,), jnp.int32)
    lanes = lax.iota(jnp.int32, L)

    def clear_hist(nbkt):
        def clr(j, carry):
            hist[pl.ds(j * L, L)] = jnp.zeros((L,), jnp.int32)
            return carry
        lax.fori_loop(0, nbkt // L, clr, 0)

    def hist_scan(start_cnt, nbkt):
        def cond(st):
            _, cnt, _ = st
            return cnt < K

        def body(st):
            b, cnt, _ = st
            nb = b - 1
            v = hist[pl.ds(nb, L)]
            return nb, cnt + v[0], cnt

        b, _, prev = lax.while_loop(
            cond, body, (jnp.int32(nbkt), start_cnt, start_cnt))
        return b, prev

    def row_body(i, carry):
        r = base + i
        pltpu.sync_copy(adj_hbm.at[r], rowbuf)

        clear_hist(H1_SIZE)

        def h1(c, carry):
            bits = lax.bitcast_convert_type(rowbuf[pl.ds(c * L, L)], jnp.int32)
            bkt = lax.shift_right_logical(bits, 19)
            plsc.addupdate_scatter(hist, [bkt], ones)
            return carry
        lax.fori_loop(0, NCHUNK, h1, 0)
        b1, c1 = hist_scan(jnp.int32(0), H1_SIZE)

        thr1 = lax.shift_left(b1, 19)

        def p2(c, cnt):
            bits = lax.bitcast_convert_type(rowbuf[pl.ds(c * L, L)], jnp.int32)
            m = bits >= thr1
            plsc.store_compressed(cbits.at[pl.ds(cnt, L)], bits, mask=m)
            plsc.store_compressed(cidx.at[pl.ds(cnt, L)], lanes + c * L,
                                  mask=m)
            return cnt + _smax(plsc.all_reduce_population_count(m))
        m_cnt = lax.fori_loop(0, NCHUNK, p2, jnp.int32(0))
        ncc = (m_cnt + L - 1) // L

        clear_hist(H2_SIZE)

        def h2(c, carry):
            bits = cbits[pl.ds(c * L, L)]
            valid = (lanes + c * L) < m_cnt
            m = valid & (lax.shift_right_logical(bits, 19) == b1)
            bkt = lax.shift_right_logical(bits, 8) & (H2_SIZE - 1)
            plsc.addupdate_scatter(hist, [bkt], ones, mask=m)
            return carry
        lax.fori_loop(0, ncc, h2, 0)
        b2, c2 = hist_scan(c1, H2_SIZE)

        clear_hist(H3_SIZE)
        pref = lax.shift_left(b1, H2_BITS) | b2

        def h3(c, carry):
            bits = cbits[pl.ds(c * L, L)]
            valid = (lanes + c * L) < m_cnt
            m = valid & (lax.shift_right_logical(bits, 8) == pref)
            bkt = bits & (H3_SIZE - 1)
            plsc.addupdate_scatter(hist, [bkt], ones, mask=m)
            return carry
        lax.fori_loop(0, ncc, h3, 0)
        b3, c3 = hist_scan(c2, H3_SIZE)

        tbits = lax.shift_left(pref, 8) | b3
        n_tie = K - c3

        def fc(c, carry):
            ngt, neq = carry
            bits = cbits[pl.ds(c * L, L)]
            iv = cidx[pl.ds(c * L, L)]
            valid = (lanes + c * L) < m_cnt
            mgt = valid & (bits > tbits)
            meq = valid & (bits == tbits)
            plsc.store_compressed(fpad.at[pl.ds(ngt, L)], iv, mask=mgt)
            plsc.store_compressed(eqidx.at[pl.ds(neq, L)], iv, mask=meq)
            ngt = ngt + _smax(plsc.all_reduce_population_count(mgt))
            neq = neq + _smax(plsc.all_reduce_population_count(meq))
            return ngt, neq
        lax.fori_loop(0, ncc, fc, (jnp.int32(0), jnp.int32(0)))

        e0 = eqidx[pl.ds(0, L)]
        plsc.store_compressed(fpad.at[pl.ds(c3, L)], e0,
                              mask=lanes < jnp.minimum(n_tie, L))
        e1 = eqidx[pl.ds(L, L)]
        plsc.store_compressed(
            fpad.at[pl.ds(c3 + jnp.minimum(n_tie, L), L)], e1,
            mask=lanes < (n_tie - L))

        fidx[pl.ds(0, L)] = fpad[pl.ds(0, L)]
        fidx[pl.ds(L, L)] = fpad[pl.ds(L, L)]

        pltpu.async_copy(h_hbm.at[fidx], grows, sem).wait()

        def mx(j, accs):
            return tuple(
                jnp.maximum(a, grows[j, pl.ds(f * L, L)])
                for f, a in enumerate(accs))
        accs = lax.fori_loop(
            0, K, mx,
            tuple(jnp.full((L,), -jnp.inf, jnp.float32)
                  for _ in range(F // L)))
        for f in range(F // L):
            orow[pl.ds(f * L, L)] = accs[f]
        pltpu.sync_copy(orow, out_hbm.at[r])
        return carry

    lax.fori_loop(0, nrows, row_body, 0)


def kernel(x, adj, W, b):
    h = _matmul(x, W, b.reshape(1, F))
    return _sc_topk(adj, h)


# speedup vs baseline: 7.8017x; 1.1243x over previous
import functools

import jax
import jax.numpy as jnp
from jax import lax
from jax.experimental import pallas as pl
from jax.experimental.pallas import tpu as pltpu
from jax.experimental.pallas import tpu_sc as plsc

N = 10000
IN_F = 128
F = 128
K = 32
L = 16
NC, NS = 2, 16
NW = NC * NS
RPW = -(-N // NW)
NCHUNK = N // L
H1_BITS, H2_BITS, H3_BITS = 11, 11, 8
H1_SIZE, H2_SIZE, H3_SIZE = 1 << H1_BITS, 1 << H2_BITS, 1 << H3_BITS
MM_BLOCK = 400


def _mm_body(x_ref, w_ref, b_ref, o_ref):
    o_ref[...] = lax.dot_general(
        x_ref[...], w_ref[...], (((1,), (1,)), ((), ())),
        preferred_element_type=jnp.float32) + b_ref[...]


_matmul = pl.pallas_call(
    _mm_body,
    out_shape=jax.ShapeDtypeStruct((N, F), jnp.float32),
    grid=(N // MM_BLOCK,),
    in_specs=[
        pl.BlockSpec((MM_BLOCK, IN_F), lambda i: (i, 0)),
        pl.BlockSpec((F, IN_F), lambda i: (0, 0)),
        pl.BlockSpec((1, F), lambda i: (0, 0)),
    ],
    out_specs=pl.BlockSpec((MM_BLOCK, F), lambda i: (i, 0)),
)


def _smax(v):
    return v[0]


_mesh = plsc.VectorSubcoreMesh(core_axis_name="c", subcore_axis_name="s")


@functools.partial(
    pl.kernel,
    out_type=jax.ShapeDtypeStruct((N, F), jnp.float32),
    mesh=_mesh,
    compiler_params=pltpu.CompilerParams(needs_layout_passes=False),
    scratch_types=[
        pltpu.VMEM((2, N), jnp.float32),
        pltpu.VMEM((H1_SIZE + L,), jnp.int32),
        pltpu.VMEM((N + L,), jnp.int32),
        pltpu.VMEM((N + L,), jnp.int32),
        pltpu.VMEM((N + L,), jnp.int32),
        pltpu.VMEM((K + L,), jnp.int32),
        pltpu.VMEM((K,), jnp.int32),
        pltpu.VMEM((K, F), jnp.float32),
        pltpu.VMEM((F,), jnp.float32),
        pltpu.SemaphoreType.DMA,
        pltpu.SemaphoreType.DMA((2,)),
    ],
)
def _sc_topk(adj_hbm, h_hbm, out_hbm, rowbuf, hist, cbits, cidx, eqidx,
             fpad, fidx, grows, orow, sem, rsem):
    wid = lax.axis_index("s") * NC + lax.axis_index("c")
    base = wid * RPW
    nrows = jnp.minimum(RPW, N - base)
    ones = jnp.ones((L,), jnp.int32)
    lanes = lax.iota(jnp.int32, L)

    def clear_hist(nbkt):
        def clr(j, carry):
            hist[pl.ds(j * L, L)] = jnp.zeros((L,), jnp.int32)
            return carry
        lax.fori_loop(0, nbkt // L, clr, 0, unroll=4)

    def hist_scan(start_cnt, nbkt):
        def cond(st):
            _, cnt, _ = st
            return cnt < K

        def body(st):
            b, cnt, _ = st
            nb = b - 1
            v = hist[pl.ds(nb, L)]
            return nb, cnt + v[0], cnt

        b, _, prev = lax.while_loop(
            cond, body, (jnp.int32(nbkt), start_cnt, start_cnt))
        return b, prev

    pltpu.async_copy(adj_hbm.at[pl.ds(base, 1)], rowbuf.at[pl.ds(0, 1)],
                     rsem.at[0])

    def row_body(i, carry):
        r = base + i
        bsel = i & 1
        pltpu.make_async_copy(adj_hbm.at[pl.ds(r, 1)],
                              rowbuf.at[pl.ds(bsel, 1)],
                              rsem.at[bsel]).wait()

        @pl.when(i + 1 < nrows)
        def _prefetch():
            pltpu.async_copy(adj_hbm.at[pl.ds(r + 1, 1)],
                             rowbuf.at[pl.ds(1 - bsel, 1)],
                             rsem.at[1 - bsel])

        clear_hist(H1_SIZE)

        def h1(c, carry):
            bits = lax.bitcast_convert_type(rowbuf[bsel, pl.ds(c * L, L)], jnp.int32)
            bkt = lax.shift_right_logical(bits, 19)
            plsc.addupdate_scatter(hist, [bkt], ones)
            return carry
        lax.fori_loop(0, NCHUNK, h1, 0, unroll=4)
        b1, c1 = hist_scan(jnp.int32(0), H1_SIZE)

        thr1 = lax.shift_left(b1, 19)

        def p2(c, cnt):
            bits = lax.bitcast_convert_type(rowbuf[bsel, pl.ds(c * L, L)], jnp.int32)
            m = bits >= thr1
            plsc.store_compressed(cbits.at[pl.ds(cnt, L)], bits, mask=m)
            plsc.store_compressed(cidx.at[pl.ds(cnt, L)], lanes + c * L,
                                  mask=m)
            return cnt + _smax(plsc.all_reduce_population_count(m))
        m_cnt = lax.fori_loop(0, NCHUNK, p2, jnp.int32(0), unroll=4)
        ncc = (m_cnt + L - 1) // L

        clear_hist(H2_SIZE)

        def h2(c, carry):
            bits = cbits[pl.ds(c * L, L)]
            valid = (lanes + c * L) < m_cnt
            m = valid & (lax.shift_right_logical(bits, 19) == b1)
            bkt = lax.shift_right_logical(bits, 8) & (H2_SIZE - 1)
            plsc.addupdate_scatter(hist, [bkt], ones, mask=m)
            return carry
        lax.fori_loop(0, ncc, h2, 0)
        b2, c2 = hist_scan(c1, H2_SIZE)

        clear_hist(H3_SIZE)
        pref = lax.shift_left(b1, H2_BITS) | b2

        def h3(c, carry):
            bits = cbits[pl.ds(c * L, L)]
            valid = (lanes + c * L) < m_cnt
            m = valid & (lax.shift_right_logical(bits, 8) == pref)
            bkt = bits & (H3_SIZE - 1)
            plsc.addupdate_scatter(hist, [bkt], ones, mask=m)
            return carry
        lax.fori_loop(0, ncc, h3, 0)
        b3, c3 = hist_scan(c2, H3_SIZE)

        tbits = lax.shift_left(pref, 8) | b3
        n_tie = K - c3

        def fc(c, carry):
            ngt, neq = carry
            bits = cbits[pl.ds(c * L, L)]
            iv = cidx[pl.ds(c * L, L)]
            valid = (lanes + c * L) < m_cnt
            mgt = valid & (bits > tbits)
            meq = valid & (bits == tbits)
            plsc.store_compressed(fpad.at[pl.ds(ngt, L)], iv, mask=mgt)
            plsc.store_compressed(eqidx.at[pl.ds(neq, L)], iv, mask=meq)
            ngt = ngt + _smax(plsc.all_reduce_population_count(mgt))
            neq = neq + _smax(plsc.all_reduce_population_count(meq))
            return ngt, neq
        lax.fori_loop(0, ncc, fc, (jnp.int32(0), jnp.int32(0)))

        e0 = eqidx[pl.ds(0, L)]
        plsc.store_compressed(fpad.at[pl.ds(c3, L)], e0,
                              mask=lanes < jnp.minimum(n_tie, L))
        e1 = eqidx[pl.ds(L, L)]
        plsc.store_compressed(
            fpad.at[pl.ds(c3 + jnp.minimum(n_tie, L), L)], e1,
            mask=lanes < (n_tie - L))

        fidx[pl.ds(0, L)] = fpad[pl.ds(0, L)]
        fidx[pl.ds(L, L)] = fpad[pl.ds(L, L)]

        pltpu.async_copy(h_hbm.at[fidx], grows, sem).wait()

        def mx(j, accs):
            return tuple(
                jnp.maximum(a, grows[j, pl.ds(f * L, L)])
                for f, a in enumerate(accs))
        accs = lax.fori_loop(
            0, K, mx,
            tuple(jnp.full((L,), -jnp.inf, jnp.float32)
                  for _ in range(F // L)))
        for f in range(F // L):
            orow[pl.ds(f * L, L)] = accs[f]
        pltpu.sync_copy(orow, out_hbm.at[r])
        return carry

    lax.fori_loop(0, nrows, row_body, 0)


def kernel(x, adj, W, b):
    h = _matmul(x, W, b.reshape(1, F))
    return _sc_topk(adj, h)
